# baseline (device time: 235869 ns/iter reference)
import jax
import jax.numpy as jnp
from jax import lax
from jax.experimental import pallas as pl
from jax.experimental.pallas import tpu as pltpu

N_DEV = 4
SQ = 2048
D_MODEL = 1024
H_LOC = 8
DH = 128
BLK = 64
N_PHASE = 4
GROUPS = SQ // (BLK * N_PHASE)
P_SEQ = GROUPS * BLK
SCALE = 0.08838834764831843


def _attn_body(x_ref, wq_ref, k_ref, v_ref, o_ref):
    xb = x_ref[0].astype(jnp.bfloat16)
    wq = wq_ref[...].astype(jnp.bfloat16)
    q = jnp.dot(xb, wq, preferred_element_type=jnp.float32)
    k = k_ref[0, 0].astype(jnp.bfloat16)
    s = lax.dot_general(
        q.astype(jnp.bfloat16), k,
        (((1,), (1,)), ((), ())),
        preferred_element_type=jnp.float32,
    ) * SCALE
    m = jnp.max(s, axis=-1, keepdims=True)
    w = jnp.exp(s - m)
    w = w / jnp.sum(w, axis=-1, keepdims=True)
    ctx = jnp.dot(
        w.astype(jnp.bfloat16), v_ref[0, 0].astype(jnp.bfloat16),
        preferred_element_type=jnp.float32,
    )
    o_ref[0, 0, :, :] = ctx.astype(o_ref.dtype)


def _ar_body(ctx_ref, wo_ref, out_ref, comm_ref, send_sems, recv_sems):
    my = lax.axis_index("i")
    left = (my + N_DEV - 1) % N_DEV
    right = (my + 1) % N_DEV

    barrier = pltpu.get_barrier_semaphore()
    for nbr in (left, right):
        pl.semaphore_signal(
            barrier, inc=1, device_id=(nbr,),
            device_id_type=pl.DeviceIdType.MESH,
        )
    pl.semaphore_wait(barrier, 2)

    partial = jnp.dot(
        ctx_ref[...], wo_ref[...], preferred_element_type=jnp.float32
    )
    comm_ref[0, :, :] = partial.astype(jnp.bfloat16)
    out_ref[0, :, :] = partial

    for h in range(N_DEV - 1):
        rdma = pltpu.make_async_remote_copy(
            src_ref=comm_ref.at[h],
            dst_ref=comm_ref.at[h + 1],
            send_sem=send_sems.at[h],
            recv_sem=recv_sems.at[h + 1],
            device_id=(right,),
            device_id_type=pl.DeviceIdType.MESH,
        )
        rdma.start()
        rdma.wait()
        out_ref[0, :, :] = (
            out_ref[0, :, :] + comm_ref[h + 1, :, :].astype(jnp.float32)
        )


def kernel(x, Wq, K_ext, V_ext, Wo):
    my = lax.axis_index("i")
    f_loc = H_LOC * DH
    wq_loc = lax.dynamic_slice(Wq, (0, my * f_loc), (D_MODEL, f_loc))
    wo_loc = lax.dynamic_slice(Wo, (my * f_loc, 0), (f_loc, D_MODEL))

    xp = x[0].reshape(GROUPS, N_PHASE, BLK, D_MODEL).transpose(1, 0, 2, 3)
    xp = xp.reshape(N_PHASE, P_SEQ, D_MODEL)
    kp = K_ext[0].reshape(GROUPS, N_PHASE, BLK, H_LOC, DH)
    kp = kp.transpose(3, 1, 0, 2, 4).reshape(H_LOC, N_PHASE, P_SEQ, DH)
    vp = V_ext[0].reshape(GROUPS, N_PHASE, BLK, H_LOC, DH)
    vp = vp.transpose(3, 1, 0, 2, 4).reshape(H_LOC, N_PHASE, P_SEQ, DH)

    ctx_p = pl.pallas_call(
        _attn_body,
        grid=(H_LOC, N_PHASE),
        in_specs=[
            pl.BlockSpec((1, P_SEQ, D_MODEL), lambda h, p: (p, 0, 0)),
            pl.BlockSpec((D_MODEL, DH), lambda h, p: (0, h)),
            pl.BlockSpec((1, 1, P_SEQ, DH), lambda h, p: (h, p, 0, 0)),
            pl.BlockSpec((1, 1, P_SEQ, DH), lambda h, p: (h, p, 0, 0)),
        ],
        out_specs=pl.BlockSpec((1, 1, P_SEQ, DH), lambda h, p: (h, p, 0, 0)),
        out_shape=jax.ShapeDtypeStruct(
            (H_LOC, N_PHASE, P_SEQ, DH), jnp.bfloat16
        ),
    )(xp, wq_loc, kp, vp)

    ctx = ctx_p.reshape(H_LOC, N_PHASE, GROUPS, BLK, DH)
    ctx = ctx.transpose(2, 1, 3, 0, 4).reshape(SQ, f_loc)

    out = pl.pallas_call(
        _ar_body,
        in_specs=[
            pl.BlockSpec(memory_space=pltpu.VMEM),
            pl.BlockSpec(memory_space=pltpu.VMEM),
        ],
        out_specs=pl.BlockSpec(memory_space=pltpu.VMEM),
        out_shape=jax.ShapeDtypeStruct((1, SQ, D_MODEL), jnp.float32),
        scratch_shapes=[
            pltpu.VMEM((N_DEV, SQ, D_MODEL), jnp.bfloat16),
            pltpu.SemaphoreType.DMA((N_DEV,)),
            pltpu.SemaphoreType.DMA((N_DEV,)),
        ],
        compiler_params=pltpu.CompilerParams(collective_id=0),
    )(ctx, wo_loc.astype(jnp.bfloat16))

    return out


# device time: 139573 ns/iter; 1.6899x vs baseline; 1.6899x over previous
import jax
import jax.numpy as jnp
from jax import lax
from jax.experimental import pallas as pl
from jax.experimental.pallas import tpu as pltpu

N_DEV = 4
SQ = 2048
D_MODEL = 1024
H_LOC = 8
DH = 128
BLK = 64
N_PHASE = 4
GROUPS = SQ // (BLK * N_PHASE)
P_SEQ = GROUPS * BLK
SCALE = 0.08838834764831843


def _attn_body(x_ref, wq_ref, k_ref, v_ref, o_ref):
    xb = x_ref[0].astype(jnp.bfloat16)
    wq = wq_ref[...].astype(jnp.bfloat16)
    q = jnp.dot(xb, wq, preferred_element_type=jnp.float32)
    k = k_ref[0, 0].astype(jnp.bfloat16)
    s = lax.dot_general(
        q.astype(jnp.bfloat16), k,
        (((1,), (1,)), ((), ())),
        preferred_element_type=jnp.float32,
    ) * SCALE
    m = jnp.max(s, axis=-1, keepdims=True)
    w = jnp.exp(s - m)
    w = w / jnp.sum(w, axis=-1, keepdims=True)
    ctx = jnp.dot(
        w.astype(jnp.bfloat16), v_ref[0, 0].astype(jnp.bfloat16),
        preferred_element_type=jnp.float32,
    )
    o_ref[0, 0, :, :] = ctx.astype(o_ref.dtype)


C_ROWS = SQ // N_DEV


def _ar_body(
    ctx_ref, wo_ref, out_ref,
    part_ref, rs_buf, ag_src, ag_buf,
    rs_send, rs_recv, ag_send, ag_recv,
):
    my = lax.axis_index("i")

    barrier = pltpu.get_barrier_semaphore()
    for k in (1, 2, 3):
        pl.semaphore_signal(
            barrier, inc=1, device_id=((my + k) % N_DEV,),
            device_id_type=pl.DeviceIdType.MESH,
        )
    pl.semaphore_wait(barrier, 3)

    wo = wo_ref[...]

    rs_rdmas = []
    for k in (1, 2, 3):
        t = (my + k) % N_DEV
        chunk = jnp.dot(
            ctx_ref[pl.ds(t * C_ROWS, C_ROWS), :], wo,
            preferred_element_type=jnp.float32,
        )
        part_ref[k - 1, :, :] = chunk.astype(jnp.bfloat16)
        r = pltpu.make_async_remote_copy(
            src_ref=part_ref.at[k - 1],
            dst_ref=rs_buf.at[k - 1],
            send_sem=rs_send.at[k - 1],
            recv_sem=rs_recv.at[k - 1],
            device_id=(t,),
            device_id_type=pl.DeviceIdType.MESH,
        )
        r.start()
        rs_rdmas.append(r)

    red = jnp.dot(
        ctx_ref[pl.ds(my * C_ROWS, C_ROWS), :], wo,
        preferred_element_type=jnp.float32,
    )
    for k in (1, 2, 3):
        rs_rdmas[k - 1].wait_recv()
        red = red + rs_buf[k - 1, :, :].astype(jnp.float32)

    ag_src[:, :] = red.astype(jnp.bfloat16)
    ag_rdmas = []
    for k in (1, 2, 3):
        t = (my + k) % N_DEV
        r = pltpu.make_async_remote_copy(
            src_ref=ag_src,
            dst_ref=ag_buf.at[k - 1],
            send_sem=ag_send.at[k - 1],
            recv_sem=ag_recv.at[k - 1],
            device_id=(t,),
            device_id_type=pl.DeviceIdType.MESH,
        )
        r.start()
        ag_rdmas.append(r)

    out_ref[0, pl.ds(my * C_ROWS, C_ROWS), :] = red
    for k in (1, 2, 3):
        ag_rdmas[k - 1].wait_recv()
        src_dev = (my + N_DEV - k) % N_DEV
        out_ref[0, pl.ds(src_dev * C_ROWS, C_ROWS), :] = (
            ag_buf[k - 1, :, :].astype(jnp.float32)
        )

    for k in (1, 2, 3):
        rs_rdmas[k - 1].wait_send()
        ag_rdmas[k - 1].wait_send()


def kernel(x, Wq, K_ext, V_ext, Wo):
    my = lax.axis_index("i")
    f_loc = H_LOC * DH
    wq_loc = lax.dynamic_slice(Wq, (0, my * f_loc), (D_MODEL, f_loc))
    wo_loc = lax.dynamic_slice(Wo, (my * f_loc, 0), (f_loc, D_MODEL))

    xp = x[0].reshape(GROUPS, N_PHASE, BLK, D_MODEL).transpose(1, 0, 2, 3)
    xp = xp.reshape(N_PHASE, P_SEQ, D_MODEL)
    kp = K_ext[0].reshape(GROUPS, N_PHASE, BLK, H_LOC, DH)
    kp = kp.transpose(3, 1, 0, 2, 4).reshape(H_LOC, N_PHASE, P_SEQ, DH)
    vp = V_ext[0].reshape(GROUPS, N_PHASE, BLK, H_LOC, DH)
    vp = vp.transpose(3, 1, 0, 2, 4).reshape(H_LOC, N_PHASE, P_SEQ, DH)

    ctx_p = pl.pallas_call(
        _attn_body,
        grid=(H_LOC, N_PHASE),
        in_specs=[
            pl.BlockSpec((1, P_SEQ, D_MODEL), lambda h, p: (p, 0, 0)),
            pl.BlockSpec((D_MODEL, DH), lambda h, p: (0, h)),
            pl.BlockSpec((1, 1, P_SEQ, DH), lambda h, p: (h, p, 0, 0)),
            pl.BlockSpec((1, 1, P_SEQ, DH), lambda h, p: (h, p, 0, 0)),
        ],
        out_specs=pl.BlockSpec((1, 1, P_SEQ, DH), lambda h, p: (h, p, 0, 0)),
        out_shape=jax.ShapeDtypeStruct(
            (H_LOC, N_PHASE, P_SEQ, DH), jnp.bfloat16
        ),
    )(xp, wq_loc, kp, vp)

    ctx = ctx_p.reshape(H_LOC, N_PHASE, GROUPS, BLK, DH)
    ctx = ctx.transpose(2, 1, 3, 0, 4).reshape(SQ, f_loc)

    out = pl.pallas_call(
        _ar_body,
        in_specs=[
            pl.BlockSpec(memory_space=pltpu.VMEM),
            pl.BlockSpec(memory_space=pltpu.VMEM),
        ],
        out_specs=pl.BlockSpec(memory_space=pltpu.VMEM),
        out_shape=jax.ShapeDtypeStruct((1, SQ, D_MODEL), jnp.float32),
        scratch_shapes=[
            pltpu.VMEM((3, SQ // N_DEV, D_MODEL), jnp.bfloat16),
            pltpu.VMEM((3, SQ // N_DEV, D_MODEL), jnp.bfloat16),
            pltpu.VMEM((SQ // N_DEV, D_MODEL), jnp.bfloat16),
            pltpu.VMEM((3, SQ // N_DEV, D_MODEL), jnp.bfloat16),
            pltpu.SemaphoreType.DMA((3,)),
            pltpu.SemaphoreType.DMA((3,)),
            pltpu.SemaphoreType.DMA((3,)),
            pltpu.SemaphoreType.DMA((3,)),
        ],
        compiler_params=pltpu.CompilerParams(collective_id=0),
    )(ctx, wo_loc.astype(jnp.bfloat16))

    return out


# device time: 125285 ns/iter; 1.8827x vs baseline; 1.1140x over previous
import jax
import jax.numpy as jnp
from jax import lax
from jax.experimental import pallas as pl
from jax.experimental.pallas import tpu as pltpu

N_DEV = 4
SQ = 2048
D_MODEL = 1024
H_LOC = 8
DH = 128
BLK = 64
N_PHASE = 4
GROUPS = SQ // (BLK * N_PHASE)
P_SEQ = GROUPS * BLK
SCALE = 0.08838834764831843


def _body(
    xp_ref, wq_ref, k_ref, v_ref, wo_ref, out_ref,
    ctx_acc, part_ref, rs_buf, ag_src, ag_buf,
    rs_send, rs_recv, ag_send, ag_recv,
):
    p = pl.program_id(0)
    h = pl.program_id(1)
    my = lax.axis_index("i")

    @pl.when((p == 0) & (h == 0))
    def _barrier():
        barrier = pltpu.get_barrier_semaphore()
        for k in (1, 2, 3):
            pl.semaphore_signal(
                barrier, inc=1, device_id=((my + k) % N_DEV,),
                device_id_type=pl.DeviceIdType.MESH,
            )
        pl.semaphore_wait(barrier, 3)

    q = jnp.dot(xp_ref[0], wq_ref[...], preferred_element_type=jnp.float32)
    s = lax.dot_general(
        q.astype(jnp.bfloat16), k_ref[0, 0],
        (((1,), (1,)), ((), ())),
        preferred_element_type=jnp.float32,
    ) * SCALE
    w = jnp.exp(s)
    denom = jnp.sum(w, axis=-1, keepdims=True)
    ctx = jnp.dot(
        w.astype(jnp.bfloat16), v_ref[0, 0],
        preferred_element_type=jnp.float32,
    ) / denom
    ctx_acc[:, pl.ds(h * DH, DH)] = ctx.astype(jnp.bfloat16)

    @pl.when(h == H_LOC - 1)
    def _project_and_send():
        partial = jnp.dot(
            ctx_acc[...], wo_ref[...], preferred_element_type=jnp.float32
        )
        part_ref[p, :, :] = partial.astype(jnp.bfloat16)
        kk = (p + N_DEV - my) % N_DEV
        for k in (1, 2, 3):
            @pl.when(kk == k)
            def _send():
                r = pltpu.make_async_remote_copy(
                    src_ref=part_ref.at[p],
                    dst_ref=rs_buf.at[k - 1],
                    send_sem=rs_send.at[k - 1],
                    recv_sem=rs_recv.at[k - 1],
                    device_id=(p,),
                    device_id_type=pl.DeviceIdType.MESH,
                )
                r.start()

    @pl.when((p == N_PHASE - 1) & (h == H_LOC - 1))
    def _tail():
        red = part_ref[my, :, :].astype(jnp.float32)
        for j in (0, 1, 2):
            rr = pltpu.make_async_remote_copy(
                src_ref=part_ref.at[0], dst_ref=rs_buf.at[j],
                send_sem=rs_send.at[j], recv_sem=rs_recv.at[j],
                device_id=(my,), device_id_type=pl.DeviceIdType.MESH,
            )
            rr.wait_recv()
            red = red + rs_buf[j, :, :].astype(jnp.float32)

        ag_src[:, :] = red.astype(jnp.bfloat16)
        ag_rdmas = []
        for k in (1, 2, 3):
            t = (my + k) % N_DEV
            r = pltpu.make_async_remote_copy(
                src_ref=ag_src,
                dst_ref=ag_buf.at[k - 1],
                send_sem=ag_send.at[k - 1],
                recv_sem=ag_recv.at[k - 1],
                device_id=(t,),
                device_id_type=pl.DeviceIdType.MESH,
            )
            r.start()
            ag_rdmas.append(r)

        for g in range(GROUPS):
            out_ref[0, pl.ds(g * N_PHASE * BLK + my * BLK, BLK), :] = (
                red[g * BLK:(g + 1) * BLK, :]
            )
        for k in (1, 2, 3):
            ag_rdmas[k - 1].wait_recv()
            src_dev = (my + N_DEV - k) % N_DEV
            chunk = ag_buf[k - 1, :, :].astype(jnp.float32)
            for g in range(GROUPS):
                out_ref[0, pl.ds(g * N_PHASE * BLK + src_dev * BLK, BLK), :] = (
                    chunk[g * BLK:(g + 1) * BLK, :]
                )

        for j in (0, 1, 2):
            ds = pltpu.make_async_remote_copy(
                src_ref=part_ref.at[0], dst_ref=rs_buf.at[j],
                send_sem=rs_send.at[j], recv_sem=rs_recv.at[j],
                device_id=(my,), device_id_type=pl.DeviceIdType.MESH,
            )
            ds.wait_send()
            ag_rdmas[j].wait_send()


def kernel(x, Wq, K_ext, V_ext, Wo):
    my = lax.axis_index("i")
    f_loc = H_LOC * DH
    wq_loc = lax.dynamic_slice(Wq, (0, my * f_loc), (D_MODEL, f_loc))
    wq_loc = wq_loc.astype(jnp.bfloat16)
    wo_loc = lax.dynamic_slice(Wo, (my * f_loc, 0), (f_loc, D_MODEL))
    wo_loc = wo_loc.astype(jnp.bfloat16)

    xb = x[0].astype(jnp.bfloat16)
    xp = xb.reshape(GROUPS, N_PHASE, BLK, D_MODEL).transpose(1, 0, 2, 3)
    xp = xp.reshape(N_PHASE, P_SEQ, D_MODEL)
    kb = K_ext[0].astype(jnp.bfloat16)
    kp = kb.reshape(GROUPS, N_PHASE, BLK, H_LOC, DH)
    kp = kp.transpose(3, 1, 0, 2, 4).reshape(H_LOC, N_PHASE, P_SEQ, DH)
    vb = V_ext[0].astype(jnp.bfloat16)
    vp = vb.reshape(GROUPS, N_PHASE, BLK, H_LOC, DH)
    vp = vp.transpose(3, 1, 0, 2, 4).reshape(H_LOC, N_PHASE, P_SEQ, DH)

    out = pl.pallas_call(
        _body,
        grid=(N_PHASE, H_LOC),
        in_specs=[
            pl.BlockSpec((1, P_SEQ, D_MODEL), lambda p, h: (p, 0, 0)),
            pl.BlockSpec((D_MODEL, DH), lambda p, h: (0, h)),
            pl.BlockSpec((1, 1, P_SEQ, DH), lambda p, h: (h, p, 0, 0)),
            pl.BlockSpec((1, 1, P_SEQ, DH), lambda p, h: (h, p, 0, 0)),
            pl.BlockSpec((f_loc, D_MODEL), lambda p, h: (0, 0)),
        ],
        out_specs=pl.BlockSpec((1, SQ, D_MODEL), lambda p, h: (0, 0, 0)),
        out_shape=jax.ShapeDtypeStruct((1, SQ, D_MODEL), jnp.float32),
        scratch_shapes=[
            pltpu.VMEM((P_SEQ, f_loc), jnp.bfloat16),
            pltpu.VMEM((N_PHASE, P_SEQ, D_MODEL), jnp.bfloat16),
            pltpu.VMEM((3, P_SEQ, D_MODEL), jnp.bfloat16),
            pltpu.VMEM((P_SEQ, D_MODEL), jnp.bfloat16),
            pltpu.VMEM((3, P_SEQ, D_MODEL), jnp.bfloat16),
            pltpu.SemaphoreType.DMA((3,)),
            pltpu.SemaphoreType.DMA((3,)),
            pltpu.SemaphoreType.DMA((3,)),
            pltpu.SemaphoreType.DMA((3,)),
        ],
        compiler_params=pltpu.CompilerParams(collective_id=0),
    )(xp, wq_loc, kp, vp, wo_loc)

    return out
